# baseline (speedup 1.0000x reference)
import jax
import jax.numpy as jnp
from jax.experimental import pallas as pl
from jax.experimental.pallas import tpu as pltpu

N_NODES = 100000
N_EDGES = 1600000
D_FEAT = 128
D_EDGE = 16
D_CTX = 128
D_OUT = 128

GRID = 10
BV = N_NODES // GRID
BE = N_EDGES // GRID


def _body(ctx_ref, v_ref, e_ref, w_ref, we_ref, b_ref, out_ref, vacc, eacc):
    i = pl.program_id(0)

    @pl.when(i == 0)
    def _init():
        vacc[...] = jnp.zeros_like(vacc)
        eacc[...] = jnp.zeros_like(eacc)

    ones = jnp.ones((1, BV), dtype=jnp.float32)
    vacc[...] += jnp.dot(ones, v_ref[...], preferred_element_type=jnp.float32)
    eacc[...] += jnp.sum(e_ref[...], axis=1, keepdims=True)

    @pl.when(i == GRID - 1)
    def _fini():
        v_mean = vacc[...] / N_NODES
        x = jnp.concatenate([ctx_ref[...], v_mean], axis=1)
        e_contrib = jnp.sum(eacc[...] * we_ref[...], axis=0,
                            keepdims=True) / N_EDGES
        out_ref[...] = (jnp.dot(x, w_ref[...],
                                preferred_element_type=jnp.float32)
                        + e_contrib + b_ref[...])


def kernel(context, vertex_data, edge_data, W, b):
    et = edge_data.T
    b2 = b.reshape(1, D_OUT)
    w_main = W[: D_CTX + D_FEAT]
    w_edge = W[D_CTX + D_FEAT:]
    out = pl.pallas_call(
        _body,
        grid=(GRID,),
        in_specs=[
            pl.BlockSpec((1, D_CTX), lambda i: (0, 0)),
            pl.BlockSpec((BV, D_FEAT), lambda i: (i, 0)),
            pl.BlockSpec((D_EDGE, BE), lambda i: (0, i)),
            pl.BlockSpec((D_CTX + D_FEAT, D_OUT), lambda i: (0, 0)),
            pl.BlockSpec((D_EDGE, D_OUT), lambda i: (0, 0)),
            pl.BlockSpec((1, D_OUT), lambda i: (0, 0)),
        ],
        out_specs=pl.BlockSpec((1, D_OUT), lambda i: (0, 0)),
        out_shape=jax.ShapeDtypeStruct((1, D_OUT), jnp.float32),
        scratch_shapes=[
            pltpu.VMEM((1, D_FEAT), jnp.float32),
            pltpu.VMEM((D_EDGE, 1), jnp.float32),
        ],
    )(context, vertex_data, et, w_main, w_edge, b2)
    return out


# speedup vs baseline: 1.0028x; 1.0028x over previous
import jax
import jax.numpy as jnp
from jax.experimental import pallas as pl
from jax.experimental.pallas import tpu as pltpu

N_NODES = 100000
N_EDGES = 1600000
D_FEAT = 128
D_EDGE = 16
D_CTX = 128
D_OUT = 128

NCH = 50
CHV = N_NODES // NCH
CHE = N_EDGES // NCH
NBUF = 4


def _body(ctx_ref, v_hbm, e_hbm, w_ref, we_ref, b_ref, out_ref,
          vbuf, ebuf, vacc, eacc, vsem, esem):
    def start(k, slot):
        pltpu.make_async_copy(v_hbm.at[pl.ds(k * CHV, CHV), :],
                              vbuf.at[slot], vsem.at[slot]).start()
        pltpu.make_async_copy(e_hbm.at[:, pl.ds(k * CHE, CHE)],
                              ebuf.at[slot], esem.at[slot]).start()

    for s in range(NBUF):
        start(s, s)

    vacc[...] = jnp.zeros_like(vacc)
    eacc[...] = jnp.zeros_like(eacc)
    ones = jnp.ones((1, CHV), dtype=jnp.float32)

    for k in range(NCH):
        slot = k % NBUF
        pltpu.make_async_copy(v_hbm.at[pl.ds(k * CHV, CHV), :],
                              vbuf.at[slot], vsem.at[slot]).wait()
        pltpu.make_async_copy(e_hbm.at[:, pl.ds(k * CHE, CHE)],
                              ebuf.at[slot], esem.at[slot]).wait()
        vacc[...] += jnp.dot(ones, vbuf[slot],
                             preferred_element_type=jnp.float32)
        eacc[...] += jnp.sum(ebuf[slot], axis=1, keepdims=True)
        if k + NBUF < NCH:
            start(k + NBUF, slot)

    v_mean = vacc[...] / N_NODES
    x = jnp.concatenate([ctx_ref[...], v_mean], axis=1)
    e_contrib = jnp.sum(eacc[...] * we_ref[...], axis=0,
                        keepdims=True) / N_EDGES
    out_ref[...] = (jnp.dot(x, w_ref[...], preferred_element_type=jnp.float32)
                    + e_contrib + b_ref[...])


def kernel(context, vertex_data, edge_data, W, b):
    et = edge_data.T
    b2 = b.reshape(1, D_OUT)
    w_main = W[: D_CTX + D_FEAT]
    w_edge = W[D_CTX + D_FEAT:]
    out = pl.pallas_call(
        _body,
        grid=(1,),
        in_specs=[
            pl.BlockSpec((1, D_CTX), lambda i: (0, 0)),
            pl.BlockSpec(memory_space=pltpu.HBM),
            pl.BlockSpec(memory_space=pltpu.HBM),
            pl.BlockSpec((D_CTX + D_FEAT, D_OUT), lambda i: (0, 0)),
            pl.BlockSpec((D_EDGE, D_OUT), lambda i: (0, 0)),
            pl.BlockSpec((1, D_OUT), lambda i: (0, 0)),
        ],
        out_specs=pl.BlockSpec((1, D_OUT), lambda i: (0, 0)),
        out_shape=jax.ShapeDtypeStruct((1, D_OUT), jnp.float32),
        scratch_shapes=[
            pltpu.VMEM((NBUF, CHV, D_FEAT), jnp.float32),
            pltpu.VMEM((NBUF, D_EDGE, CHE), jnp.float32),
            pltpu.VMEM((1, D_FEAT), jnp.float32),
            pltpu.VMEM((D_EDGE, 1), jnp.float32),
            pltpu.SemaphoreType.DMA((NBUF,)),
            pltpu.SemaphoreType.DMA((NBUF,)),
        ],
    )(context, vertex_data, et, w_main, w_edge, b2)
    return out
